# baseline (speedup 1.0000x reference)
import functools

import jax
import jax.numpy as jnp
from jax.experimental import pallas as pl

N = 10000
E = 320000
HD = 128


def _epilogue_body(hs_ref, agg_ref, h_ref, deg_ref, out_ref):
    hs = hs_ref[...]
    agg = agg_ref[...]
    h = h_ref[...]
    deg = deg_ref[...]
    h_new = jnp.where(deg > 0, hs + agg, h)
    out_ref[...] = h + jnp.maximum(h_new, 0.0)


def _epilogue(hs, agg, h, deg):
    n, d = h.shape
    blk = 1000
    grid = (n // blk,)
    return pl.pallas_call(
        _epilogue_body,
        grid=grid,
        in_specs=[
            pl.BlockSpec((blk, d), lambda i: (i, 0)),
            pl.BlockSpec((blk, d), lambda i: (i, 0)),
            pl.BlockSpec((blk, d), lambda i: (i, 0)),
            pl.BlockSpec((blk, 1), lambda i: (i, 0)),
        ],
        out_specs=pl.BlockSpec((blk, d), lambda i: (i, 0)),
        out_shape=jax.ShapeDtypeStruct((n, d), h.dtype),
    )(hs, agg, h, deg)


def _layer(h, src, dst, g, W_self, W_func, a):
    hd = h.shape[1]
    a1 = a[0, :hd]
    a2 = a[0, hd:2 * hd]
    h_s = h @ W_self.T
    z = h @ W_func.T
    fs = z @ a1
    fd = z @ a2
    e = jax.nn.leaky_relu(fs[src] + fd[dst] + g, 0.01)
    e_max = jax.ops.segment_max(e, dst, num_segments=N)
    e_max = jnp.where(jnp.isfinite(e_max), e_max, 0.0)
    e_exp = jnp.exp(e - e_max[dst])
    denom = jax.ops.segment_sum(e_exp, dst, num_segments=N)
    alpha = e_exp / jnp.where(denom[dst] > 0, denom[dst], 1.0)
    agg = jax.ops.segment_sum(alpha[:, None] * z[src], dst, num_segments=N)
    deg = jax.ops.segment_sum(jnp.ones((dst.shape[0],), h.dtype), dst, num_segments=N)
    return _epilogue(h_s, agg, h, deg[:, None])


def kernel(x, edge_index, e_w, snorm_n, W_self_h0, W_func_h0, a_h0,
           W_self_h1, W_func_h1, a_h1, W_self_2, W_func_2, a_2):
    src = edge_index[0]
    dst = edge_index[1]
    c0 = jnp.stack([a_h0[0, 2 * HD:2 * HD + 64].sum(), a_h0[0, 2 * HD + 64:].sum()])
    c1 = jnp.stack([a_h1[0, 2 * HD:2 * HD + 64].sum(), a_h1[0, 2 * HD + 64:].sum()])
    D2 = 2 * HD
    c2 = jnp.stack([a_2[0, 2 * D2:2 * D2 + HD].sum(), a_2[0, 2 * D2 + HD:].sum()])
    g0 = e_w @ c0
    g1 = e_w @ c1
    g2 = e_w @ c2
    h0 = _layer(x, src, dst, g0, W_self_h0, W_func_h0, a_h0)
    h1 = _layer(x, src, dst, g1, W_self_h1, W_func_h1, a_h1)
    h = jnp.concatenate([h0, h1], axis=1)
    return _layer(h, src, dst, g2, W_self_2, W_func_2, a_2)


# speedup vs baseline: 7.7412x; 7.7412x over previous
import functools

import jax
import jax.numpy as jnp
from jax import lax
from jax.experimental import pallas as pl
from jax.experimental.pallas import tpu as pltpu
from jax.experimental.pallas import tpu_sc as plsc

N = 10000
E = 320000
HD = 128
D2 = 256
NP = 10240
NT = 16
EPT = E // NT
NVR = EPT // 16
GB = 80
W = 64
NB = EPT // GB
NPT = NP // NT
RR = 8
RH = NP // RR
SPT = RH // NT
HN = NP // 2
HPT = HN // NT

f32 = jnp.float32
i32 = jnp.int32


_BN = 1024


def _pre1_body(x_ref, wst_ref, wft_ref, av_ref, hs_ref, z_ref, fsfd_ref):
    xb = x_ref[...]
    for c in range(2):
        zc = jnp.dot(xb, wft_ref[c], preferred_element_type=f32)
        hs_ref[c] = jnp.dot(xb, wst_ref[c], preferred_element_type=f32)
        z_ref[2 * c] = zc[:, :W]
        z_ref[2 * c + 1] = zc[:, W:]
        fsfd_ref[0, c, :] = zc @ av_ref[c, :HD]
        fsfd_ref[1, c, :] = zc @ av_ref[c, HD:2 * HD]


def _pre1(x_pad, wst, wft, av):
    grid = (NP // _BN,)
    return pl.pallas_call(
        _pre1_body,
        grid=grid,
        in_specs=[
            pl.BlockSpec((_BN, HD), lambda i: (i, 0)),
            pl.BlockSpec((2, HD, HD), lambda i: (0, 0, 0)),
            pl.BlockSpec((2, HD, HD), lambda i: (0, 0, 0)),
            pl.BlockSpec((2, 3 * HD), lambda i: (0, 0)),
        ],
        out_specs=[
            pl.BlockSpec((2, _BN, HD), lambda i: (0, i, 0)),
            pl.BlockSpec((4, _BN, W), lambda i: (0, i, 0)),
            pl.BlockSpec((2, 2, _BN), lambda i: (0, 0, i)),
        ],
        out_shape=[
            jax.ShapeDtypeStruct((2, NP, HD), f32),
            jax.ShapeDtypeStruct((4, NP, W), f32),
            jax.ShapeDtypeStruct((2, 2, NP), f32),
        ],
    )(x_pad, wst, wft, av)


def _pre2_body(h_ref, wst_ref, wft_ref, av_ref, hs_ref, zs_ref, fsfd_ref):
    hb = h_ref[...]
    z2 = jnp.dot(hb, wft_ref[...], preferred_element_type=f32)
    hs_ref[...] = jnp.dot(hb, wst_ref[...], preferred_element_type=f32)
    for q in range(4):
        zs_ref[q] = z2[:, q * W:(q + 1) * W]
    fsfd_ref[0, :] = z2 @ av_ref[0, :D2]
    fsfd_ref[1, :] = z2 @ av_ref[0, D2:2 * D2]


def _pre2(h_cat, ws2t, wf2t, a2):
    grid = (NP // _BN,)
    return pl.pallas_call(
        _pre2_body,
        grid=grid,
        in_specs=[
            pl.BlockSpec((_BN, D2), lambda i: (i, 0)),
            pl.BlockSpec((D2, D2), lambda i: (0, 0)),
            pl.BlockSpec((D2, D2), lambda i: (0, 0)),
            pl.BlockSpec((1, 3 * D2), lambda i: (0, 0)),
        ],
        out_specs=[
            pl.BlockSpec((_BN, D2), lambda i: (i, 0)),
            pl.BlockSpec((4, _BN, W), lambda i: (0, i, 0)),
            pl.BlockSpec((2, _BN), lambda i: (0, i)),
        ],
        out_shape=[
            jax.ShapeDtypeStruct((NP, D2), f32),
            jax.ShapeDtypeStruct((4, NP, W), f32),
            jax.ShapeDtypeStruct((2, NP), f32),
        ],
    )(h_cat, ws2t, wf2t, a2)


_BE = 32000


def _g_body(ewt_ref, a0_ref, a1_ref, a2_ref, g_ref):
    w0 = ewt_ref[0, :]
    w1 = ewt_ref[1, :]
    c00 = jnp.sum(a0_ref[0, 2 * HD:2 * HD + 64])
    c01 = jnp.sum(a0_ref[0, 2 * HD + 64:])
    c10 = jnp.sum(a1_ref[0, 2 * HD:2 * HD + 64])
    c11 = jnp.sum(a1_ref[0, 2 * HD + 64:])
    c20 = jnp.sum(a2_ref[0, 2 * D2:2 * D2 + HD])
    c21 = jnp.sum(a2_ref[0, 2 * D2 + HD:])
    g_ref[0, :] = w0 * c00 + w1 * c01
    g_ref[1, :] = w0 * c10 + w1 * c11
    g_ref[2, :] = w0 * c20 + w1 * c21


def _g_kernel(ewt, a_h0, a_h1, a_2):
    grid = (E // _BE,)
    return pl.pallas_call(
        _g_body,
        grid=grid,
        in_specs=[
            pl.BlockSpec((2, _BE), lambda i: (0, i)),
            pl.BlockSpec((1, 3 * HD), lambda i: (0, 0)),
            pl.BlockSpec((1, 3 * HD), lambda i: (0, 0)),
            pl.BlockSpec((1, 3 * D2), lambda i: (0, 0)),
        ],
        out_specs=pl.BlockSpec((3, _BE), lambda i: (0, i)),
        out_shape=jax.ShapeDtypeStruct((3, E), f32),
    )(ewt, a_h0, a_h1, a_2)


def _epi1_body(x_ref, hs_ref, agg_ref, den_ref, out_ref):
    xb = x_ref[...]
    for c in range(2):
        agg_c = jnp.concatenate([agg_ref[2 * c], agg_ref[2 * c + 1]], axis=1)
        hn = jnp.where(den_ref[c][:, None] > 0, hs_ref[c] + agg_c, xb)
        out_ref[:, c * HD:(c + 1) * HD] = xb + jnp.maximum(hn, 0.0)


def _epi1(x_pad, hs, agg, den):
    grid = (NP // _BN,)
    return pl.pallas_call(
        _epi1_body,
        grid=grid,
        in_specs=[
            pl.BlockSpec((_BN, HD), lambda i: (i, 0)),
            pl.BlockSpec((2, _BN, HD), lambda i: (0, i, 0)),
            pl.BlockSpec((4, _BN, W), lambda i: (0, i, 0)),
            pl.BlockSpec((2, _BN), lambda i: (0, i)),
        ],
        out_specs=pl.BlockSpec((_BN, D2), lambda i: (i, 0)),
        out_shape=jax.ShapeDtypeStruct((NP, D2), f32),
    )(x_pad, hs, agg, den)


_BO = 1000


def _epi2_body(h_ref, hs_ref, agg_ref, den_ref, out_ref):
    hb = h_ref[...]
    aggcat = jnp.concatenate([agg_ref[0], agg_ref[1], agg_ref[2],
                              agg_ref[3]], axis=1)
    hn = jnp.where(den_ref[0] > 0, hs_ref[...] + aggcat, hb)
    out_ref[...] = hb + jnp.maximum(hn, 0.0)


def _epi2(h_cat, hs2, agg2, den2):
    grid = (N // _BO,)
    return pl.pallas_call(
        _epi2_body,
        grid=grid,
        in_specs=[
            pl.BlockSpec((_BO, D2), lambda i: (i, 0)),
            pl.BlockSpec((_BO, D2), lambda i: (i, 0)),
            pl.BlockSpec((4, _BO, W), lambda i: (0, i, 0)),
            pl.BlockSpec((2, _BO, 1), lambda i: (0, i, 0)),
        ],
        out_specs=pl.BlockSpec((_BO, D2), lambda i: (i, 0)),
        out_shape=jax.ShapeDtypeStruct((N, D2), f32),
    )(h_cat, hs2, agg2, den2[..., None])


def _seg_scan(sk, sv, op):
    lane = lax.iota(i32, 16)

    def pick(arr, idx):
        return arr.at[idx].get(mode="promise_in_bounds")

    for d in (1, 2, 4, 8):
        idx = jnp.maximum(lane - d, 0)
        kz = pick(sk, idx)
        vz = pick(sv, idx)
        ok = (kz == sk) & (lane >= d)
        sv = jnp.where(ok, op(sv, vz), sv)
    nxt = pick(sk, jnp.minimum(lane + 1, 15))
    islast = (nxt != sk) | (lane == 15)
    return sv, islast


def _sc_body(zcat, fs2, fd2, g2, src_h, dst_h, agg_out, den_out,
             srcb, dstb, ev, nb1, nb2, rows, idxb, reda, redb,
             aggS, stage, globv, sem0, sem1):
    cid = lax.axis_index("c")
    sid = lax.axis_index("s")
    ebase = pl.multiple_of(sid * EPT, EPT)
    nbase = pl.multiple_of(sid * NPT, NPT)

    pltpu.sync_copy(src_h.at[pl.ds(ebase, EPT)], srcb)
    pltpu.sync_copy(dst_h.at[pl.ds(ebase, EPT)], dstb)
    pltpu.sync_copy(g2.at[pl.ds(pl.multiple_of(cid * E + ebase, EPT), EPT)], ev)
    pltpu.sync_copy(fs2.at[pl.ds(pl.multiple_of(cid * NP, NP), NP)], nb1)
    pltpu.sync_copy(fd2.at[pl.ds(pl.multiple_of(cid * NP, NP), NP)], nb2)

    def p1(i, _):
        sl = pl.ds(i * 16, 16)
        sv = srcb[sl]
        dv = dstb[sl]
        u = plsc.load_gather(nb1, [sv]) + plsc.load_gather(nb2, [dv]) + ev[sl]
        ev[sl] = jnp.where(u >= 0, u, 0.01 * u)
        return 0
    lax.fori_loop(0, NVR, p1, 0)

    def zinit(i, _):
        nb1[pl.ds(i * 16, 16)] = jnp.full((16,), -jnp.inf, f32)
        return 0
    lax.fori_loop(0, NP // 16, zinit, 0)

    def p2(i, _):
        sl = pl.ds(i * 16, 16)
        sk, sv = plsc.sort_key_val(dstb[sl], ev[sl])
        m, islast = _seg_scan(sk, sv, jnp.maximum)
        cur = plsc.load_gather(nb1, [sk])
        plsc.store_scatter(nb1, [sk], jnp.maximum(cur, m), mask=islast)
        return 0
    lax.fori_loop(0, NVR, p2, 0)

    def _tree_reduce(buf, op):
        for h in range(RR):
            hb = h * RH
            pltpu.sync_copy(buf.at[pl.ds(hb, RH)], stage.at[sid])
            plsc.subcore_barrier()
            base = pl.multiple_of(sid * SPT, SPT)
            ssl = pl.ds(base, SPT)
            pltpu.sync_copy(stage.at[0, ssl], reda.at[pl.ds(0, SPT)])

            def rk(k, _):
                pltpu.sync_copy(stage.at[k, ssl], redb.at[pl.ds(0, SPT)])

                def rv(j, _):
                    jl = pl.ds(j * 16, 16)
                    reda[jl] = op(reda[jl], redb[jl])
                    return 0
                lax.fori_loop(0, SPT // 16, rv, 0)
                return 0
            lax.fori_loop(1, NT, rk, 0)
            pltpu.sync_copy(reda, globv.at[pl.ds(hb + base, SPT)])
            plsc.subcore_barrier()
        pltpu.sync_copy(globv, buf)

    _tree_reduce(nb1, jnp.maximum)

    def zinit2(i, _):
        nb2[pl.ds(i * 16, 16)] = jnp.zeros((16,), f32)
        return 0
    lax.fori_loop(0, NP // 16, zinit2, 0)

    def p3(i, _):
        sl = pl.ds(i * 16, 16)
        dv = dstb[sl]
        p = jnp.exp(ev[sl] - plsc.load_gather(nb1, [dv]))
        ev[sl] = p
        sk, sv = plsc.sort_key_val(dv, p)
        s, islast = _seg_scan(sk, sv, lambda a, b: a + b)
        cur = plsc.load_gather(nb2, [sk])
        plsc.store_scatter(nb2, [sk], cur + s, mask=islast)
        return 0
    lax.fori_loop(0, NVR, p3, 0)

    _tree_reduce(nb2, lambda a, b: a + b)
    pltpu.sync_copy(nb2.at[pl.ds(nbase, NPT)],
                    den_out.at[pl.ds(pl.multiple_of(cid * NP + nbase, NPT), NPT)])

    def p4(i, _):
        sl = pl.ds(i * 16, 16)
        ev[sl] = ev[sl] / plsc.load_gather(nb2, [dstb[sl]])
        return 0
    lax.fori_loop(0, NVR, p4, 0)

    def vec_pass(h, rh):
        off = pl.multiple_of((2 * cid + h) * NP, NP)
        rbase = rh * HN

        def zr(r, _):
            for k in range(W // 16):
                rows[0, r, pl.ds(k * 16, 16)] = jnp.zeros((16,), f32)
            return 0
        lax.fori_loop(0, GB, zr, 0)
        hbase = pl.multiple_of(sid * HPT, HPT)
        for q in range(HPT // GB):
            pltpu.sync_copy(rows.at[0], aggS.at[pl.ds(hbase + q * GB, GB)])
        plsc.subcore_barrier()

        def fill_and_start(j, b):
            sem = sem0 if b == 0 else sem1
            for k in range(GB // 16):
                kl = pl.ds(k * 16, 16)
                idxb[b, kl] = srcb[pl.ds(j * GB + k * 16, 16)] + off
            return pltpu.async_copy(zcat.at[idxb.at[b]], rows.at[b], sem)

        def wait_g(b):
            sem = sem0 if b == 0 else sem1
            pltpu.make_async_copy(zcat.at[idxb.at[b]], rows.at[b], sem).wait()

        def process(j, b):
            wait_g(b)

            def scale16(r16, _):
                av = ev[pl.ds(j * GB + r16 * 16, 16)]
                for rr in range(16):
                    al = av[rr]
                    r = r16 * 16 + rr
                    for k in range(W // 16):
                        kl = pl.ds(k * 16, 16)
                        rows[b, r, kl] = rows[b, r, kl] * al
                return 0
            lax.fori_loop(0, GB // 16, scale16, 0)
            for k in range(GB // 16):
                dvec = dstb[pl.ds(j * GB + k * 16, 16)]
                inrange = (dvec >= rbase) & (dvec < rbase + HN)
                dloc = jnp.where(inrange, dvec - rbase, HN)
                pltpu.sync_copy(rows.at[b, pl.ds(k * 16, 16)],
                                aggS.at[dloc], add=True)

        fill_and_start(0, 0)

        def vec_loop(jj, _):
            j0 = jj * 2
            fill_and_start(jnp.minimum(j0 + 1, NB - 1), 1)
            process(j0, 0)
            fill_and_start(jnp.minimum(j0 + 2, NB - 1), 0)
            process(j0 + 1, 1)
            return 0
        lax.fori_loop(0, NB // 2, vec_loop, 0)
        wait_g(0)

        plsc.subcore_barrier()

        for q in range(HPT // GB):
            pltpu.sync_copy(
                aggS.at[pl.ds(hbase + q * GB, GB)],
                agg_out.at[pl.ds(off + rbase + hbase + q * GB, GB)])
        plsc.subcore_barrier()

    for _h in range(2):
        for _rh in range(2):
            vec_pass(_h, _rh)


@functools.partial(
    pl.kernel,
    out_type=[
        jax.ShapeDtypeStruct((4 * NP, W), f32),
        jax.ShapeDtypeStruct((2 * NP,), f32),
    ],
    mesh=plsc.VectorSubcoreMesh(core_axis_name="c", subcore_axis_name="s",
                                num_cores=2, num_subcores=16),
    compiler_params=pltpu.CompilerParams(needs_layout_passes=False,
                                         use_tc_tiling_on_sc=False),
    scratch_types=[
        pltpu.VMEM((EPT,), i32),
        pltpu.VMEM((EPT,), i32),
        pltpu.VMEM((EPT,), f32),
        pltpu.VMEM((NP,), f32),
        pltpu.VMEM((NP,), f32),
        pltpu.VMEM((2, GB, W), f32),
        pltpu.VMEM((2, GB), i32),
        pltpu.VMEM((SPT,), f32),
        pltpu.VMEM((SPT,), f32),
        pltpu.VMEM_SHARED((HN + 8, W), f32),
        pltpu.VMEM_SHARED((NT, RH), f32),
        pltpu.VMEM_SHARED((NP,), f32),
        pltpu.SemaphoreType.DMA,
        pltpu.SemaphoreType.DMA,
    ],
)
def _sc_gat(zcat, fs2, fd2, g2, src_h, dst_h, agg_out, den_out,
            srcb, dstb, ev, nb1, nb2, rows, idxb, reda, redb,
            aggS, stage, globv, sem0, sem1):
    _sc_body(zcat, fs2, fd2, g2, src_h, dst_h, agg_out, den_out,
             srcb, dstb, ev, nb1, nb2, rows, idxb, reda, redb,
             aggS, stage, globv, sem0, sem1)


def kernel(x, edge_index, e_w, snorm_n, W_self_h0, W_func_h0, a_h0,
           W_self_h1, W_func_h1, a_h1, W_self_2, W_func_2, a_2):
    src = edge_index[0]
    dst = edge_index[1]

    x_pad = jnp.pad(x, ((0, NP - N), (0, 0)))
    wst = jnp.stack([W_self_h0.T, W_self_h1.T])
    wft = jnp.stack([W_func_h0.T, W_func_h1.T])
    av = jnp.concatenate([a_h0, a_h1], axis=0)

    hs, z, fsfd = _pre1(x_pad, wst, wft, av)
    g3 = _g_kernel(e_w.T, a_h0, a_h1, a_2)

    zcat = z.reshape(4 * NP, W)
    agg, den = _sc_gat(zcat, fsfd[0].reshape(2 * NP), fsfd[1].reshape(2 * NP),
                       g3[:2].reshape(2 * E), src, dst)
    h_cat = _epi1(x_pad, hs, agg.reshape(4, NP, W), den.reshape(2, NP))

    hs2, zs2, fsfd2 = _pre2(h_cat, W_self_2.T, W_func_2.T, a_2)
    zcat2 = zs2.reshape(4 * NP, W)
    fs2 = jnp.concatenate([fsfd2[0], fsfd2[0]])
    fd2 = jnp.concatenate([fsfd2[1], fsfd2[1]])
    g22 = jnp.concatenate([g3[2], g3[2]])
    agg2, den2 = _sc_gat(zcat2, fs2, fd2, g22, src, dst)
    return _epi2(h_cat, hs2, agg2.reshape(4, NP, W), den2.reshape(2, NP))
